# Initial kernel scaffold; baseline (speedup 1.0000x reference)
import jax
import jax.numpy as jnp
from jax.experimental import pallas as pl


def kernel(z, codebook):
    raise NotImplementedError("write your pallas kernel here")


# speedup vs baseline: 1.2640x; 1.2640x over previous
import functools

import jax
import jax.numpy as jnp
from jax import lax
from jax.experimental import pallas as pl
from jax.experimental.pallas import tpu as pltpu
from jax.experimental.pallas import tpu_sc as plsc

B, T, D, K = 16, 256, 256, 8192
N = B * T

BLOCK_N = 4096
BLOCK_K = 1024
NB = N // BLOCK_N
KB = K // BLOCK_K


def _assign_kernel(rowsum_ref, flat_ref, cb_ref, colsum_ref, iota_ref,
                   idx_ref, loss_ref, buf_a, buf_b):
    x = flat_ref[...] * -2.0
    rs = rowsum_ref[...]
    bufs = [buf_a, buf_b]

    minval = None
    minidx = None

    def epilogue(k, buf, minval, minidx):
        lo, hi = k * BLOCK_K, (k + 1) * BLOCK_K
        d = (rs + buf[...]) + colsum_ref[0:1, lo:hi]
        tile_min = jnp.min(d, axis=1, keepdims=True)
        ids = iota_ref[0:1, lo:hi]
        tile_arg = jnp.min(jnp.where(d == tile_min, ids, float(K)),
                           axis=1, keepdims=True)
        if minval is None:
            return tile_min, tile_arg
        better = tile_min < minval
        return (jnp.where(better, tile_min, minval),
                jnp.where(better, tile_arg, minidx))

    for k in range(KB):
        c = cb_ref[k * BLOCK_K:(k + 1) * BLOCK_K, :]
        mm = lax.dot_general(x, c, (((1,), (1,)), ((), ())),
                             preferred_element_type=jnp.float32)
        bufs[k % 2][...] = mm
        if k > 0:
            minval, minidx = epilogue(k - 1, bufs[(k - 1) % 2],
                                      minval, minidx)
    minval, minidx = epilogue(KB - 1, bufs[(KB - 1) % 2], minval, minidx)

    idx_ref[...] = minidx.astype(jnp.int32)
    loss_ref[0, 0] = jnp.sum(minval)


def _assign(flat, codebook, rowsum, colsum):
    return pl.pallas_call(
        _assign_kernel,
        in_specs=[
            pl.BlockSpec((N, 1), lambda: (0, 0)),
            pl.BlockSpec((N, D), lambda: (0, 0)),
            pl.BlockSpec((K, D), lambda: (0, 0)),
            pl.BlockSpec((1, K), lambda: (0, 0)),
            pl.BlockSpec((1, K), lambda: (0, 0)),
        ],
        out_specs=[
            pl.BlockSpec((N, 1), lambda: (0, 0)),
            pl.BlockSpec(memory_space=pltpu.SMEM),
        ],
        out_shape=[
            jax.ShapeDtypeStruct((N, 1), jnp.int32),
            jax.ShapeDtypeStruct((1, 1), jnp.float32),
        ],
        scratch_shapes=[
            pltpu.VMEM((N, BLOCK_K), jnp.float32),
            pltpu.VMEM((N, BLOCK_K), jnp.float32),
        ],
    )(rowsum, flat, codebook, colsum,
      jnp.arange(K, dtype=jnp.float32)[None, :])


def _make_gather():
    info = plsc.get_sparse_core_info()
    nw = info.num_cores * info.num_subcores
    b_per_w = N // nw
    mesh = plsc.VectorSubcoreMesh(core_axis_name="c", subcore_axis_name="s")

    @functools.partial(
        pl.kernel, mesh=mesh,
        out_type=jax.ShapeDtypeStruct((N, D), jnp.float32),
        scratch_types=[
            pltpu.VMEM((b_per_w,), jnp.int32),
            pltpu.VMEM((b_per_w, D), jnp.float32),
            pltpu.SemaphoreType.DMA,
        ],
    )
    def gather(table_hbm, idx_hbm, out_hbm, idx_v, rows_v, sem):
        wid = lax.axis_index("s") * info.num_cores + lax.axis_index("c")
        base = wid * b_per_w
        pltpu.sync_copy(idx_hbm.at[pl.ds(base, b_per_w)], idx_v)
        pltpu.async_copy(table_hbm.at[idx_v], rows_v, sem).wait()
        pltpu.sync_copy(rows_v, out_hbm.at[pl.ds(base, b_per_w)])

    return gather


_gather = _make_gather()


def kernel(z, codebook):
    flat = z.reshape(N, D)
    rowsum = jnp.sum(flat * flat, axis=1, keepdims=True)
    colsum = jnp.sum(codebook * codebook, axis=1)[None, :]
    idx2d, loss = _assign(flat, codebook, rowsum, colsum)
    idx = idx2d.reshape(N)
    vq_loss = (loss * (1.25 / (N * D))).reshape(())
    zq = _gather(codebook, idx).reshape(B, T, D)
    zq_st = z + lax.stop_gradient(zq - z)
    return zq_st, vq_loss, idx
